# baseline (speedup 1.0000x reference)
import functools

import jax
import jax.numpy as jnp
from jax import lax
from jax.experimental import pallas as pl
from jax.experimental.pallas import tpu as pltpu
from jax.experimental.pallas import tpu_sc as plsc

N = 10000
E = 320000
G = 256

NC = 2
NS = 16
NW = NC * NS
EPW = E // NW
C = 80
NCHUNK = EPW // C
K = 5
NSUPER = NCHUNK // K
NPAIR = NSUPER // 2
RPT = 624
RTAIL = N - NS * RPT

BR = 1000


def _leaky(t):
    return jnp.maximum(t, 0.01 * t)


@functools.cache
def _sc_scatter(F):
    mesh = plsc.VectorSubcoreMesh(core_axis_name="c", subcore_axis_name="s")

    @functools.partial(
        pl.kernel,
        out_type=jax.ShapeDtypeStruct((2 * N, F), jnp.float32),
        mesh=mesh,
        scratch_types=(
            [pltpu.VMEM((NCHUNK, C), jnp.int32),
             pltpu.VMEM((NCHUNK, C), jnp.int32),
             pltpu.VMEM((NCHUNK, C), jnp.float32)]
            + [pltpu.VMEM((C, F), jnp.float32) for _ in range(2 * K)]
            + [pltpu.VMEM_SHARED((N, F), jnp.float32),
               pltpu.SemaphoreType.DMA,
               pltpu.SemaphoreType.DMA,
               pltpu.SemaphoreType.DMA,
               pltpu.SemaphoreType.DMA]
        ),
        compiler_params=pltpu.CompilerParams(use_tc_tiling_on_sc=False),
    )
    def scat(h_hbm, src_hbm, dst_hbm, ew_hbm, zero_hbm, out_hbm, *refs):
        src_v, dst_v, ew_v = refs[0], refs[1], refs[2]
        rows_a = refs[3:3 + K]
        rows_b = refs[3 + K:3 + 2 * K]
        acc_s = refs[3 + 2 * K]
        gsem_a, gsem_b, ssem_a, ssem_b = refs[4 + 2 * K:8 + 2 * K]
        cid = lax.axis_index("c")
        sid = lax.axis_index("s")
        wid = sid * NC + cid

        pltpu.sync_copy(zero_hbm.at[pl.ds(sid * RPT, RPT)],
                        acc_s.at[pl.ds(sid * RPT, RPT)])

        @pl.when(sid == NS - 1)
        def _():
            pltpu.sync_copy(zero_hbm.at[pl.ds(NS * RPT, RTAIL)],
                            acc_s.at[pl.ds(NS * RPT, RTAIL)])

        cbase0 = wid * NCHUNK
        pltpu.sync_copy(src_hbm.at[pl.ds(cbase0, NCHUNK)], src_v)
        pltpu.sync_copy(dst_hbm.at[pl.ds(cbase0, NCHUNK)], dst_v)
        pltpu.sync_copy(ew_hbm.at[pl.ds(cbase0, NCHUNK)], ew_v)
        plsc.subcore_barrier()

        def fire_g(bufs, sem, cb):
            for j in range(K):
                pltpu.async_copy(h_hbm.at[src_v.at[cb + j]], bufs[j], sem)

        def drain_g(bufs, sem):
            for j in range(K):
                pltpu.make_async_copy(h_hbm.at[src_v.at[0]], bufs[j],
                                      sem).wait()

        def fire_s(bufs, sem, cb):
            for j in range(K):
                pltpu.async_copy(bufs[j], acc_s.at[dst_v.at[cb + j]], sem,
                                 add=True)

        def drain_s(bufs, sem):
            for j in range(K):
                pltpu.make_async_copy(bufs[j], acc_s.at[dst_v.at[0]],
                                      sem).wait()

        def compute(bufs, cb):
            for j in range(K):
                @plsc.parallel_loop(0, C, step=16, unroll=4)
                def group(gbase, j=j):
                    ew16 = ew_v[cb + j, pl.ds(gbase, 16)]
                    for e in range(16):
                        w = ew16.at[jnp.full((16,), e, jnp.int32)].get(
                            mode="promise_in_bounds")
                        for f in range(F // 16):
                            v = bufs[j][gbase + e, pl.ds(f * 16, 16)]
                            bufs[j][gbase + e, pl.ds(f * 16, 16)] = v * w

        fire_g(rows_a, gsem_a, 0)

        def pair(i, carry):
            a = 2 * i * K
            b = a + K

            @pl.when(i > 0)
            def _():
                drain_s(rows_b, ssem_b)

            fire_g(rows_b, gsem_b, b)
            drain_g(rows_a, gsem_a)
            compute(rows_a, a)
            fire_s(rows_a, ssem_a, a)
            drain_s(rows_a, ssem_a)
            fire_g(rows_a, gsem_a, a + 2 * K)
            drain_g(rows_b, gsem_b)
            compute(rows_b, b)
            fire_s(rows_b, ssem_b, b)
            return carry

        lax.fori_loop(0, NPAIR, pair, 0)
        eb = 2 * NPAIR * K
        drain_s(rows_b, ssem_b)
        drain_g(rows_a, gsem_a)
        compute(rows_a, eb)
        fire_s(rows_a, ssem_a, eb)
        drain_s(rows_a, ssem_a)
        plsc.subcore_barrier()
        pltpu.sync_copy(acc_s.at[pl.ds(sid * RPT, RPT)],
                        out_hbm.at[pl.ds(cid * N + sid * RPT, RPT)])

        @pl.when(sid == NS - 1)
        def _():
            pltpu.sync_copy(acc_s.at[pl.ds(NS * RPT, RTAIL)],
                            out_hbm.at[pl.ds(cid * N + NS * RPT, RTAIL)])

    return scat


def _mm_first(x, W):
    def body(x_ref, w_ref, o_ref):
        o_ref[...] = jnp.dot(x_ref[...], w_ref[...],
                             preferred_element_type=jnp.float32)

    Fi, Fo = W.shape
    return pl.pallas_call(
        body,
        grid=(N // BR,),
        in_specs=[pl.BlockSpec((BR, Fi), lambda i: (i, 0)),
                  pl.BlockSpec((Fi, Fo), lambda i: (0, 0))],
        out_specs=pl.BlockSpec((BR, Fo), lambda i: (i, 0)),
        out_shape=jax.ShapeDtypeStruct((N, Fo), jnp.float32),
    )(x, W)


def _fuse_layer(p, b, W):
    def body(p_ref, b_ref, w_ref, o_ref):
        s = p_ref[0] + p_ref[1] + b_ref[...]
        o_ref[...] = jnp.dot(_leaky(s), w_ref[...],
                             preferred_element_type=jnp.float32)

    F = p.shape[-1]
    Fo = W.shape[1]
    return pl.pallas_call(
        body,
        grid=(N // BR,),
        in_specs=[pl.BlockSpec((2, BR, F), lambda i: (0, i, 0)),
                  pl.BlockSpec((1, F), lambda i: (0, 0)),
                  pl.BlockSpec((F, Fo), lambda i: (0, 0))],
        out_specs=pl.BlockSpec((BR, Fo), lambda i: (i, 0)),
        out_shape=jax.ShapeDtypeStruct((N, Fo), jnp.float32),
    )(p, b, W)


def _final_conv(p, b):
    def body(p_ref, b_ref, conv_ref, st_ref):
        s = p_ref[0] + p_ref[1] + b_ref[...]
        conv_ref[...] = s

        @pl.when(pl.program_id(0) == 0)
        def _():
            st_ref[...] = jnp.zeros_like(st_ref)

        st_ref[0:1, :] = st_ref[0:1, :] + jnp.sum(s, axis=0, keepdims=True)
        st_ref[1:2, :] = st_ref[1:2, :] + jnp.sum(s * s, axis=0,
                                                  keepdims=True)

    F = p.shape[-1]
    return pl.pallas_call(
        body,
        grid=(N // BR,),
        in_specs=[pl.BlockSpec((2, BR, F), lambda i: (0, i, 0)),
                  pl.BlockSpec((1, F), lambda i: (0, 0))],
        out_specs=[pl.BlockSpec((BR, F), lambda i: (i, 0)),
                   pl.BlockSpec((8, F), lambda i: (0, 0))],
        out_shape=[jax.ShapeDtypeStruct((N, F), jnp.float32),
                   jax.ShapeDtypeStruct((8, F), jnp.float32)],
    )(p, b)


def _bn_pool(conv, stats, gamma, beta, batch3d):
    def body(c_ref, st_ref, g_ref, b_ref, bt_ref, o_ref):
        mean = st_ref[0:1, :] * (1.0 / N)
        var = st_ref[1:2, :] * (1.0 / N) - mean * mean
        inv = lax.rsqrt(var + 1e-5)
        s = (c_ref[...] - mean) * inv * g_ref[...] + b_ref[...]
        s = _leaky(s)
        sel = (bt_ref[0] ==
               lax.broadcasted_iota(jnp.int32, (G, 1), 0)).astype(jnp.float32)
        part = jnp.dot(sel, s, preferred_element_type=jnp.float32)

        @pl.when(pl.program_id(0) == 0)
        def _():
            o_ref[...] = jnp.zeros_like(o_ref)

        o_ref[...] = o_ref[...] + part

    F = conv.shape[-1]
    return pl.pallas_call(
        body,
        grid=(N // BR,),
        in_specs=[pl.BlockSpec((BR, F), lambda i: (i, 0)),
                  pl.BlockSpec((8, F), lambda i: (0, 0)),
                  pl.BlockSpec((1, F), lambda i: (0, 0)),
                  pl.BlockSpec((1, F), lambda i: (0, 0)),
                  pl.BlockSpec((1, 1, BR), lambda i: (i, 0, 0))],
        out_specs=pl.BlockSpec((G, F), lambda i: (0, 0)),
        out_shape=jax.ShapeDtypeStruct((G, F), jnp.float32),
    )(conv, stats, gamma, beta, batch3d)


def _mlp(pooled, w1, b1, w2, b2, w3, b3):
    def body(p_ref, w1r, b1r, w2r, b2r, w3r, b3r, o_ref):
        a = _leaky(jnp.dot(p_ref[...], w1r[...],
                           preferred_element_type=jnp.float32) + b1r[...])
        a = _leaky(jnp.dot(a, w2r[...],
                           preferred_element_type=jnp.float32) + b2r[...])
        a = _leaky(jnp.dot(a, w3r[...],
                           preferred_element_type=jnp.float32) + b3r[...])
        o_ref[...] = a

    H = w1.shape[1]
    return pl.pallas_call(
        body,
        in_specs=[pl.BlockSpec(pooled.shape, lambda: (0, 0)),
                  pl.BlockSpec(w1.shape, lambda: (0, 0)),
                  pl.BlockSpec(b1.shape, lambda: (0, 0)),
                  pl.BlockSpec(w2.shape, lambda: (0, 0)),
                  pl.BlockSpec(b2.shape, lambda: (0, 0)),
                  pl.BlockSpec(w3.shape, lambda: (0, 0)),
                  pl.BlockSpec(b3.shape, lambda: (0, 0))],
        out_specs=pl.BlockSpec((G, H), lambda: (0, 0)),
        out_shape=jax.ShapeDtypeStruct((G, H), jnp.float32),
    )(pooled, w1, b1, w2, b2, w3, b3)


def _pad2(a, r, c):
    return jnp.pad(a, ((0, r - a.shape[0]), (0, c - a.shape[1])))


def kernel(x, edge_index, edge_weigth, batch, W1, b1, W2, b2, W3, b3, W4, b4,
           gamma, beta, fcw1, fcb1, fcw2, fcb2, fcw3, fcb3):
    src = edge_index[0]
    dst = edge_index[1]

    W4p = _pad2(W4, 64, 64)
    b4p = jnp.pad(b4, (0, 14)).reshape(1, 64)
    gammap = jnp.pad(gamma, (0, 14)).reshape(1, 64)
    betap = jnp.pad(beta, (0, 14)).reshape(1, 64)
    fw1 = _pad2(fcw1, 64, 128)
    fb1 = jnp.pad(fcb1, (0, 98)).reshape(1, 128)
    fw2 = _pad2(fcw2, 128, 128)
    fb2 = jnp.pad(fcb2, (0, 108)).reshape(1, 128)
    fw3 = _pad2(fcw3, 128, 128)
    fb3 = jnp.pad(fcb3, (0, 126)).reshape(1, 128)

    src2 = src.reshape(E // C, C)
    dst2 = dst.reshape(E // C, C)
    ew2 = edge_weigth.reshape(E // C, C)

    def scat(h, F):
        zeros = jnp.zeros((N, F), jnp.float32)
        p = _sc_scatter(F)(h, src2, dst2, ew2, zeros)
        return p.reshape(2, N, F)

    h1 = _mm_first(x, W1)
    p1 = scat(h1, 16)
    h2 = _fuse_layer(p1, b1.reshape(1, 16), W2)
    p2 = scat(h2, 32)
    h3 = _fuse_layer(p2, b2.reshape(1, 32), W3)
    p3 = scat(h3, 64)
    h4 = _fuse_layer(p3, b3.reshape(1, 64), W4p)
    p4 = scat(h4, 64)
    conv, stats = _final_conv(p4, b4p)
    pooled = _bn_pool(conv, stats, gammap, betap, batch.reshape(N // BR, 1, BR))
    out = _mlp(pooled, fw1, fb1, fw2, fb2, fw3, fb3)
    return out[:, :2]


# speedup vs baseline: 1.2039x; 1.2039x over previous
import functools

import jax
import jax.numpy as jnp
from jax import lax
from jax.experimental import pallas as pl
from jax.experimental.pallas import tpu as pltpu
from jax.experimental.pallas import tpu_sc as plsc

N = 10000
E = 320000
G = 256

NC = 2
NS = 16
NW = NC * NS
EPW = E // NW
C = 80
NCHUNK = EPW // C
K = 5
NSUPER = NCHUNK // K
NPAIR = NSUPER // 2
RPT = 624
RTAIL = N - NS * RPT

BR = 1000


def _leaky(t):
    return jnp.maximum(t, 0.01 * t)


@functools.cache
def _sc_scatter(F):
    mesh = plsc.VectorSubcoreMesh(core_axis_name="c", subcore_axis_name="s")

    @functools.partial(
        pl.kernel,
        out_type=jax.ShapeDtypeStruct((2 * N, F), jnp.float32),
        mesh=mesh,
        scratch_types=(
            [pltpu.VMEM((NCHUNK, C), jnp.int32),
             pltpu.VMEM((NCHUNK, C), jnp.int32),
             pltpu.VMEM((NCHUNK, C), jnp.float32)]
            + [pltpu.VMEM((C, F), jnp.float32) for _ in range(2 * K)]
            + [pltpu.VMEM_SHARED((N, F), jnp.float32),
               pltpu.SemaphoreType.DMA,
               pltpu.SemaphoreType.DMA,
               pltpu.SemaphoreType.DMA,
               pltpu.SemaphoreType.DMA]
        ),
        compiler_params=pltpu.CompilerParams(use_tc_tiling_on_sc=False),
    )
    def scat(h_hbm, src_hbm, dst_hbm, ew_hbm, zero_hbm, out_hbm, *refs):
        src_v, dst_v, ew_v = refs[0], refs[1], refs[2]
        rows_a = refs[3:3 + K]
        rows_b = refs[3 + K:3 + 2 * K]
        acc_s = refs[3 + 2 * K]
        gsem_a, gsem_b, ssem_a, ssem_b = refs[4 + 2 * K:8 + 2 * K]
        cid = lax.axis_index("c")
        sid = lax.axis_index("s")
        wid = sid * NC + cid

        pltpu.sync_copy(zero_hbm.at[pl.ds(sid * RPT, RPT)],
                        acc_s.at[pl.ds(sid * RPT, RPT)])

        @pl.when(sid == NS - 1)
        def _():
            pltpu.sync_copy(zero_hbm.at[pl.ds(NS * RPT, RTAIL)],
                            acc_s.at[pl.ds(NS * RPT, RTAIL)])

        cbase0 = wid * NCHUNK
        pltpu.sync_copy(src_hbm.at[pl.ds(cbase0, NCHUNK)], src_v)
        pltpu.sync_copy(dst_hbm.at[pl.ds(cbase0, NCHUNK)], dst_v)
        pltpu.sync_copy(ew_hbm.at[pl.ds(cbase0, NCHUNK)], ew_v)
        plsc.subcore_barrier()

        def fire_g(bufs, sem, cb):
            for j in range(K):
                pltpu.async_copy(h_hbm.at[src_v.at[cb + j]], bufs[j], sem)

        def drain_g(bufs, sem):
            for j in range(K):
                pltpu.make_async_copy(h_hbm.at[src_v.at[0]], bufs[j],
                                      sem).wait()

        def fire_s(bufs, sem, cb):
            for j in range(K):
                pltpu.async_copy(bufs[j], acc_s.at[dst_v.at[cb + j]], sem,
                                 add=True)

        def drain_s(bufs, sem):
            for j in range(K):
                pltpu.make_async_copy(bufs[j], acc_s.at[dst_v.at[0]],
                                      sem).wait()

        def compute(bufs, cb):
            for j in range(K):
                @plsc.parallel_loop(0, C, step=16, unroll=1)
                def group(gbase, j=j):
                    ew16 = ew_v[cb + j, pl.ds(gbase, 16)]
                    for e in range(16):
                        w = ew16.at[jnp.full((16,), e, jnp.int32)].get(
                            mode="promise_in_bounds")
                        for f in range(F // 16):
                            v = bufs[j][gbase + e, pl.ds(f * 16, 16)]
                            bufs[j][gbase + e, pl.ds(f * 16, 16)] = v * w

        fire_g(rows_a, gsem_a, 0)

        def pair(i, carry):
            a = 2 * i * K
            b = a + K

            @pl.when(i > 0)
            def _():
                drain_s(rows_b, ssem_b)

            fire_g(rows_b, gsem_b, b)
            drain_g(rows_a, gsem_a)
            compute(rows_a, a)
            fire_s(rows_a, ssem_a, a)
            drain_s(rows_a, ssem_a)
            fire_g(rows_a, gsem_a, a + 2 * K)
            drain_g(rows_b, gsem_b)
            compute(rows_b, b)
            fire_s(rows_b, ssem_b, b)
            return carry

        lax.fori_loop(0, NPAIR, pair, 0)
        eb = 2 * NPAIR * K
        drain_s(rows_b, ssem_b)
        drain_g(rows_a, gsem_a)
        compute(rows_a, eb)
        fire_s(rows_a, ssem_a, eb)
        drain_s(rows_a, ssem_a)
        plsc.subcore_barrier()
        pltpu.sync_copy(acc_s.at[pl.ds(sid * RPT, RPT)],
                        out_hbm.at[pl.ds(cid * N + sid * RPT, RPT)])

        @pl.when(sid == NS - 1)
        def _():
            pltpu.sync_copy(acc_s.at[pl.ds(NS * RPT, RTAIL)],
                            out_hbm.at[pl.ds(cid * N + NS * RPT, RTAIL)])

    return scat


def _mm_first(x, W):
    def body(x_ref, w_ref, o_ref):
        o_ref[...] = jnp.dot(x_ref[...], w_ref[...],
                             preferred_element_type=jnp.float32)

    Fi, Fo = W.shape
    return pl.pallas_call(
        body,
        grid=(N // BR,),
        in_specs=[pl.BlockSpec((BR, Fi), lambda i: (i, 0)),
                  pl.BlockSpec((Fi, Fo), lambda i: (0, 0))],
        out_specs=pl.BlockSpec((BR, Fo), lambda i: (i, 0)),
        out_shape=jax.ShapeDtypeStruct((N, Fo), jnp.float32),
    )(x, W)


def _fuse_layer(p, b, W):
    def body(p_ref, b_ref, w_ref, o_ref):
        s = p_ref[0] + p_ref[1] + b_ref[...]
        o_ref[...] = jnp.dot(_leaky(s), w_ref[...],
                             preferred_element_type=jnp.float32)

    F = p.shape[-1]
    Fo = W.shape[1]
    return pl.pallas_call(
        body,
        grid=(N // BR,),
        in_specs=[pl.BlockSpec((2, BR, F), lambda i: (0, i, 0)),
                  pl.BlockSpec((1, F), lambda i: (0, 0)),
                  pl.BlockSpec((F, Fo), lambda i: (0, 0))],
        out_specs=pl.BlockSpec((BR, Fo), lambda i: (i, 0)),
        out_shape=jax.ShapeDtypeStruct((N, Fo), jnp.float32),
    )(p, b, W)


def _final_conv(p, b):
    def body(p_ref, b_ref, conv_ref, st_ref):
        s = p_ref[0] + p_ref[1] + b_ref[...]
        conv_ref[...] = s

        @pl.when(pl.program_id(0) == 0)
        def _():
            st_ref[...] = jnp.zeros_like(st_ref)

        st_ref[0:1, :] = st_ref[0:1, :] + jnp.sum(s, axis=0, keepdims=True)
        st_ref[1:2, :] = st_ref[1:2, :] + jnp.sum(s * s, axis=0,
                                                  keepdims=True)

    F = p.shape[-1]
    return pl.pallas_call(
        body,
        grid=(N // BR,),
        in_specs=[pl.BlockSpec((2, BR, F), lambda i: (0, i, 0)),
                  pl.BlockSpec((1, F), lambda i: (0, 0))],
        out_specs=[pl.BlockSpec((BR, F), lambda i: (i, 0)),
                   pl.BlockSpec((8, F), lambda i: (0, 0))],
        out_shape=[jax.ShapeDtypeStruct((N, F), jnp.float32),
                   jax.ShapeDtypeStruct((8, F), jnp.float32)],
    )(p, b)


def _bn_pool(conv, stats, gamma, beta, batch3d):
    def body(c_ref, st_ref, g_ref, b_ref, bt_ref, o_ref):
        mean = st_ref[0:1, :] * (1.0 / N)
        var = st_ref[1:2, :] * (1.0 / N) - mean * mean
        inv = lax.rsqrt(var + 1e-5)
        s = (c_ref[...] - mean) * inv * g_ref[...] + b_ref[...]
        s = _leaky(s)
        sel = (bt_ref[0] ==
               lax.broadcasted_iota(jnp.int32, (G, 1), 0)).astype(jnp.float32)
        part = jnp.dot(sel, s, preferred_element_type=jnp.float32)

        @pl.when(pl.program_id(0) == 0)
        def _():
            o_ref[...] = jnp.zeros_like(o_ref)

        o_ref[...] = o_ref[...] + part

    F = conv.shape[-1]
    return pl.pallas_call(
        body,
        grid=(N // BR,),
        in_specs=[pl.BlockSpec((BR, F), lambda i: (i, 0)),
                  pl.BlockSpec((8, F), lambda i: (0, 0)),
                  pl.BlockSpec((1, F), lambda i: (0, 0)),
                  pl.BlockSpec((1, F), lambda i: (0, 0)),
                  pl.BlockSpec((1, 1, BR), lambda i: (i, 0, 0))],
        out_specs=pl.BlockSpec((G, F), lambda i: (0, 0)),
        out_shape=jax.ShapeDtypeStruct((G, F), jnp.float32),
    )(conv, stats, gamma, beta, batch3d)


def _mlp(pooled, w1, b1, w2, b2, w3, b3):
    def body(p_ref, w1r, b1r, w2r, b2r, w3r, b3r, o_ref):
        a = _leaky(jnp.dot(p_ref[...], w1r[...],
                           preferred_element_type=jnp.float32) + b1r[...])
        a = _leaky(jnp.dot(a, w2r[...],
                           preferred_element_type=jnp.float32) + b2r[...])
        a = _leaky(jnp.dot(a, w3r[...],
                           preferred_element_type=jnp.float32) + b3r[...])
        o_ref[...] = a

    H = w1.shape[1]
    return pl.pallas_call(
        body,
        in_specs=[pl.BlockSpec(pooled.shape, lambda: (0, 0)),
                  pl.BlockSpec(w1.shape, lambda: (0, 0)),
                  pl.BlockSpec(b1.shape, lambda: (0, 0)),
                  pl.BlockSpec(w2.shape, lambda: (0, 0)),
                  pl.BlockSpec(b2.shape, lambda: (0, 0)),
                  pl.BlockSpec(w3.shape, lambda: (0, 0)),
                  pl.BlockSpec(b3.shape, lambda: (0, 0))],
        out_specs=pl.BlockSpec((G, H), lambda: (0, 0)),
        out_shape=jax.ShapeDtypeStruct((G, H), jnp.float32),
    )(pooled, w1, b1, w2, b2, w3, b3)


def _pad2(a, r, c):
    return jnp.pad(a, ((0, r - a.shape[0]), (0, c - a.shape[1])))


def kernel(x, edge_index, edge_weigth, batch, W1, b1, W2, b2, W3, b3, W4, b4,
           gamma, beta, fcw1, fcb1, fcw2, fcb2, fcw3, fcb3):
    src = edge_index[0]
    dst = edge_index[1]

    W4p = _pad2(W4, 64, 64)
    b4p = jnp.pad(b4, (0, 14)).reshape(1, 64)
    gammap = jnp.pad(gamma, (0, 14)).reshape(1, 64)
    betap = jnp.pad(beta, (0, 14)).reshape(1, 64)
    fw1 = _pad2(fcw1, 64, 128)
    fb1 = jnp.pad(fcb1, (0, 98)).reshape(1, 128)
    fw2 = _pad2(fcw2, 128, 128)
    fb2 = jnp.pad(fcb2, (0, 108)).reshape(1, 128)
    fw3 = _pad2(fcw3, 128, 128)
    fb3 = jnp.pad(fcb3, (0, 126)).reshape(1, 128)

    src2 = src.reshape(E // C, C)
    dst2 = dst.reshape(E // C, C)
    ew2 = edge_weigth.reshape(E // C, C)

    def scat(h, F):
        zeros = jnp.zeros((N, F), jnp.float32)
        p = _sc_scatter(F)(h, src2, dst2, ew2, zeros)
        return p.reshape(2, N, F)

    h1 = _mm_first(x, W1)
    p1 = scat(h1, 16)
    h2 = _fuse_layer(p1, b1.reshape(1, 16), W2)
    p2 = scat(h2, 32)
    h3 = _fuse_layer(p2, b2.reshape(1, 32), W3)
    p3 = scat(h3, 64)
    h4 = _fuse_layer(p3, b3.reshape(1, 64), W4p)
    p4 = scat(h4, 64)
    conv, stats = _final_conv(p4, b4p)
    pooled = _bn_pool(conv, stats, gammap, betap, batch.reshape(N // BR, 1, BR))
    out = _mlp(pooled, fw1, fb1, fw2, fb2, fw3, fb3)
    return out[:, :2]


# speedup vs baseline: 1.2709x; 1.0557x over previous
import functools

import jax
import jax.numpy as jnp
from jax import lax
from jax.experimental import pallas as pl
from jax.experimental.pallas import tpu as pltpu
from jax.experimental.pallas import tpu_sc as plsc

N = 10000
E = 320000
G = 256

NC = 2
NS = 16
NW = NC * NS
EPW = E // NW
C = 80
NCHUNK = EPW // C
K = 5
NSUPER = NCHUNK // K
NPAIR = NSUPER // 2
RPT = 624
RTAIL = N - NS * RPT

BR = 2000
NB = N // BR


def _leaky(t):
    return jnp.maximum(t, 0.01 * t)


@functools.cache
def _sc_scatter(F):
    mesh = plsc.VectorSubcoreMesh(core_axis_name="c", subcore_axis_name="s")

    @functools.partial(
        pl.kernel,
        out_type=jax.ShapeDtypeStruct((2 * N, F), jnp.float32),
        mesh=mesh,
        scratch_types=(
            [pltpu.VMEM((NCHUNK, C), jnp.int32),
             pltpu.VMEM((NCHUNK, C), jnp.int32),
             pltpu.VMEM((NCHUNK, C), jnp.float32)]
            + [pltpu.VMEM((C, F), jnp.float32) for _ in range(2 * K)]
            + [pltpu.VMEM_SHARED((N, F), jnp.float32),
               pltpu.SemaphoreType.DMA,
               pltpu.SemaphoreType.DMA,
               pltpu.SemaphoreType.DMA,
               pltpu.SemaphoreType.DMA]
        ),
        compiler_params=pltpu.CompilerParams(use_tc_tiling_on_sc=False),
    )
    def scat(h_hbm, src_hbm, dst_hbm, ew_hbm, zero_hbm, out_hbm, *refs):
        src_v, dst_v, ew_v = refs[0], refs[1], refs[2]
        rows_a = refs[3:3 + K]
        rows_b = refs[3 + K:3 + 2 * K]
        acc_s = refs[3 + 2 * K]
        gsem_a, gsem_b, ssem_a, ssem_b = refs[4 + 2 * K:8 + 2 * K]
        cid = lax.axis_index("c")
        sid = lax.axis_index("s")
        wid = sid * NC + cid

        pltpu.sync_copy(zero_hbm.at[pl.ds(sid * RPT, RPT)],
                        acc_s.at[pl.ds(sid * RPT, RPT)])

        @pl.when(sid == NS - 1)
        def _():
            pltpu.sync_copy(zero_hbm.at[pl.ds(NS * RPT, RTAIL)],
                            acc_s.at[pl.ds(NS * RPT, RTAIL)])

        cbase0 = wid * NCHUNK
        pltpu.sync_copy(src_hbm.at[pl.ds(cbase0, NCHUNK)], src_v)
        pltpu.sync_copy(dst_hbm.at[pl.ds(cbase0, NCHUNK)], dst_v)
        pltpu.sync_copy(ew_hbm.at[pl.ds(cbase0, NCHUNK)], ew_v)
        plsc.subcore_barrier()

        def fire_g(bufs, sem, cb):
            for j in range(K):
                pltpu.async_copy(h_hbm.at[src_v.at[cb + j]], bufs[j], sem)

        def drain_g(bufs, sem):
            for j in range(K):
                pltpu.make_async_copy(h_hbm.at[src_v.at[0]], bufs[j],
                                      sem).wait()

        def fire_s(bufs, sem, cb):
            for j in range(K):
                pltpu.async_copy(bufs[j], acc_s.at[dst_v.at[cb + j]], sem,
                                 add=True)

        def drain_s(bufs, sem):
            for j in range(K):
                pltpu.make_async_copy(bufs[j], acc_s.at[dst_v.at[0]],
                                      sem).wait()

        def compute(bufs, cb):
            for j in range(K):
                @plsc.parallel_loop(0, C, step=16, unroll=1)
                def group(gbase, j=j):
                    ew16 = ew_v[cb + j, pl.ds(gbase, 16)]
                    for e in range(16):
                        w = ew16.at[jnp.full((16,), e, jnp.int32)].get(
                            mode="promise_in_bounds")
                        for f in range(F // 16):
                            v = bufs[j][gbase + e, pl.ds(f * 16, 16)]
                            bufs[j][gbase + e, pl.ds(f * 16, 16)] = v * w

        fire_g(rows_a, gsem_a, 0)

        def pair(i, carry):
            a = 2 * i * K
            b = a + K

            @pl.when(i > 0)
            def _():
                drain_s(rows_b, ssem_b)

            fire_g(rows_b, gsem_b, b)
            drain_g(rows_a, gsem_a)
            compute(rows_a, a)
            fire_s(rows_a, ssem_a, a)
            drain_s(rows_a, ssem_a)
            fire_g(rows_a, gsem_a, a + 2 * K)
            drain_g(rows_b, gsem_b)
            compute(rows_b, b)
            fire_s(rows_b, ssem_b, b)
            return carry

        lax.fori_loop(0, NPAIR, pair, 0)
        eb = 2 * NPAIR * K
        drain_s(rows_b, ssem_b)
        drain_g(rows_a, gsem_a)
        compute(rows_a, eb)
        fire_s(rows_a, ssem_a, eb)
        drain_s(rows_a, ssem_a)
        plsc.subcore_barrier()
        pltpu.sync_copy(acc_s.at[pl.ds(sid * RPT, RPT)],
                        out_hbm.at[pl.ds(cid * N + sid * RPT, RPT)])

        @pl.when(sid == NS - 1)
        def _():
            pltpu.sync_copy(acc_s.at[pl.ds(NS * RPT, RTAIL)],
                            out_hbm.at[pl.ds(cid * N + NS * RPT, RTAIL)])

    return scat


def _mm_first(x, W):
    def body(x_ref, w_ref, o_ref):
        o_ref[...] = jnp.dot(x_ref[...], w_ref[...],
                             preferred_element_type=jnp.float32)

    Fi, Fo = W.shape
    return pl.pallas_call(
        body,
        grid=(N // BR,),
        in_specs=[pl.BlockSpec((BR, Fi), lambda i: (i, 0)),
                  pl.BlockSpec((Fi, Fo), lambda i: (0, 0))],
        out_specs=pl.BlockSpec((BR, Fo), lambda i: (i, 0)),
        out_shape=jax.ShapeDtypeStruct((N, Fo), jnp.float32),
    )(x, W)


def _fuse_layer(p, b, W):
    def body(p_ref, b_ref, w_ref, o_ref):
        s = p_ref[0] + p_ref[1] + b_ref[...]
        o_ref[...] = jnp.dot(_leaky(s), w_ref[...],
                             preferred_element_type=jnp.float32)

    F = p.shape[-1]
    Fo = W.shape[1]
    return pl.pallas_call(
        body,
        grid=(N // BR,),
        in_specs=[pl.BlockSpec((2, BR, F), lambda i: (0, i, 0)),
                  pl.BlockSpec((1, F), lambda i: (0, 0)),
                  pl.BlockSpec((F, Fo), lambda i: (0, 0))],
        out_specs=pl.BlockSpec((BR, Fo), lambda i: (i, 0)),
        out_shape=jax.ShapeDtypeStruct((N, Fo), jnp.float32),
    )(p, b, W)


def _tail(p, b, gamma, beta, batch3d, w1, b1, w2, b2, w3, b3):
    F = p.shape[-1]
    H = w1.shape[1]

    def body(p_ref, b_ref, g_ref, be_ref, bt_ref, w1r, b1r, w2r, b2r, w3r,
             b3r, o_ref, conv_s, st_s, pool_s):
        i = pl.program_id(0)

        @pl.when(i == 0)
        def _():
            st_s[...] = jnp.zeros_like(st_s)
            pool_s[...] = jnp.zeros_like(pool_s)

        @pl.when(i < NB)
        def _():
            s = p_ref[0] + p_ref[1] + b_ref[...]
            conv_s[pl.ds(i * BR, BR), :] = s
            st_s[0:1, :] = st_s[0:1, :] + jnp.sum(s, axis=0, keepdims=True)
            st_s[1:2, :] = st_s[1:2, :] + jnp.sum(s * s, axis=0,
                                                  keepdims=True)

        @pl.when((i >= NB) & (i < 2 * NB))
        def _():
            k = i - NB
            mean = st_s[0:1, :] * (1.0 / N)
            var = st_s[1:2, :] * (1.0 / N) - mean * mean
            inv = lax.rsqrt(var + 1e-5)
            s = conv_s[pl.ds(k * BR, BR), :]
            s = _leaky((s - mean) * inv * g_ref[...] + be_ref[...])
            sel = (bt_ref[0] ==
                   lax.broadcasted_iota(jnp.int32, (G, 1), 0)
                   ).astype(jnp.float32)
            pool_s[...] = pool_s[...] + jnp.dot(
                sel, s, preferred_element_type=jnp.float32)

        @pl.when(i == 2 * NB)
        def _():
            a = _leaky(jnp.dot(pool_s[...], w1r[...],
                               preferred_element_type=jnp.float32) + b1r[...])
            a = _leaky(jnp.dot(a, w2r[...],
                               preferred_element_type=jnp.float32) + b2r[...])
            a = _leaky(jnp.dot(a, w3r[...],
                               preferred_element_type=jnp.float32) + b3r[...])
            o_ref[...] = a

    clamp1 = lambda i: (0, jnp.minimum(i, NB - 1), 0)
    clamp2 = lambda i: (jnp.clip(i - NB, 0, NB - 1), 0, 0)
    const2 = lambda i: (0, 0)
    return pl.pallas_call(
        body,
        grid=(2 * NB + 1,),
        in_specs=[pl.BlockSpec((2, BR, F), clamp1),
                  pl.BlockSpec((1, F), const2),
                  pl.BlockSpec((1, F), const2),
                  pl.BlockSpec((1, F), const2),
                  pl.BlockSpec((1, 1, BR), clamp2),
                  pl.BlockSpec(w1.shape, const2),
                  pl.BlockSpec(b1.shape, const2),
                  pl.BlockSpec(w2.shape, const2),
                  pl.BlockSpec(b2.shape, const2),
                  pl.BlockSpec(w3.shape, const2),
                  pl.BlockSpec(b3.shape, const2)],
        out_specs=pl.BlockSpec((G, H), const2),
        out_shape=jax.ShapeDtypeStruct((G, H), jnp.float32),
        scratch_shapes=[pltpu.VMEM((N, F), jnp.float32),
                        pltpu.VMEM((8, F), jnp.float32),
                        pltpu.VMEM((G, F), jnp.float32)],
    )(p, b, gamma, beta, batch3d, w1, b1, w2, b2, w3, b3)


def _pad2(a, r, c):
    return jnp.pad(a, ((0, r - a.shape[0]), (0, c - a.shape[1])))


def kernel(x, edge_index, edge_weigth, batch, W1, b1, W2, b2, W3, b3, W4, b4,
           gamma, beta, fcw1, fcb1, fcw2, fcb2, fcw3, fcb3):
    src = edge_index[0]
    dst = edge_index[1]

    W4p = _pad2(W4, 64, 64)
    b4p = jnp.pad(b4, (0, 14)).reshape(1, 64)
    gammap = jnp.pad(gamma, (0, 14)).reshape(1, 64)
    betap = jnp.pad(beta, (0, 14)).reshape(1, 64)
    fw1 = _pad2(fcw1, 64, 128)
    fb1 = jnp.pad(fcb1, (0, 98)).reshape(1, 128)
    fw2 = _pad2(fcw2, 128, 128)
    fb2 = jnp.pad(fcb2, (0, 108)).reshape(1, 128)
    fw3 = _pad2(fcw3, 128, 128)
    fb3 = jnp.pad(fcb3, (0, 126)).reshape(1, 128)

    src2 = src.reshape(E // C, C)
    dst2 = dst.reshape(E // C, C)
    ew2 = edge_weigth.reshape(E // C, C)

    def scat(h, F):
        zeros = jnp.zeros((N, F), jnp.float32)
        p = _sc_scatter(F)(h, src2, dst2, ew2, zeros)
        return p.reshape(2, N, F)

    h1 = _mm_first(x, W1)
    p1 = scat(h1, 16)
    h2 = _fuse_layer(p1, b1.reshape(1, 16), W2)
    p2 = scat(h2, 32)
    h3 = _fuse_layer(p2, b2.reshape(1, 32), W3)
    p3 = scat(h3, 64)
    h4 = _fuse_layer(p3, b3.reshape(1, 64), W4p)
    p4 = scat(h4, 64)
    out = _tail(p4, b4p, gammap, betap, batch.reshape(NB, 1, BR),
                fw1, fb1, fw2, fb2, fw3, fb3)
    return out[:, :2]


# speedup vs baseline: 1.2762x; 1.0041x over previous
import functools

import jax
import jax.numpy as jnp
from jax import lax
from jax.experimental import pallas as pl
from jax.experimental.pallas import tpu as pltpu
from jax.experimental.pallas import tpu_sc as plsc

N = 10000
E = 320000
G = 256

NC = 2
NS = 16
NW = NC * NS
EPW = E // NW
C = 80
NCHUNK = EPW // C
K = 5
NSUPER = NCHUNK // K
NPAIR = NSUPER // 2
RPT = 624
RTAIL = N - NS * RPT

BR = 2000
NB = N // BR


def _leaky(t):
    return jnp.maximum(t, 0.01 * t)


@functools.cache
def _sc_scatter(F):
    K = 25 if F <= 16 else 5
    NSUPER = NCHUNK // K
    NPAIR = NSUPER // 2
    mesh = plsc.VectorSubcoreMesh(core_axis_name="c", subcore_axis_name="s")

    @functools.partial(
        pl.kernel,
        out_type=jax.ShapeDtypeStruct((2 * N, F), jnp.float32),
        mesh=mesh,
        scratch_types=(
            [pltpu.VMEM((NCHUNK, C), jnp.int32),
             pltpu.VMEM((NCHUNK, C), jnp.int32),
             pltpu.VMEM((NCHUNK, C), jnp.float32)]
            + [pltpu.VMEM((C, F), jnp.float32) for _ in range(2 * K)]
            + [pltpu.VMEM_SHARED((N, F), jnp.float32),
               pltpu.SemaphoreType.DMA,
               pltpu.SemaphoreType.DMA,
               pltpu.SemaphoreType.DMA,
               pltpu.SemaphoreType.DMA]
        ),
        compiler_params=pltpu.CompilerParams(use_tc_tiling_on_sc=False),
    )
    def scat(h_hbm, src_hbm, dst_hbm, ew_hbm, zero_hbm, out_hbm, *refs):
        src_v, dst_v, ew_v = refs[0], refs[1], refs[2]
        rows_a = refs[3:3 + K]
        rows_b = refs[3 + K:3 + 2 * K]
        acc_s = refs[3 + 2 * K]
        gsem_a, gsem_b, ssem_a, ssem_b = refs[4 + 2 * K:8 + 2 * K]
        cid = lax.axis_index("c")
        sid = lax.axis_index("s")
        wid = sid * NC + cid

        cbase0 = wid * NCHUNK
        pltpu.sync_copy(src_hbm.at[pl.ds(cbase0, NCHUNK)], src_v)
        pltpu.sync_copy(dst_hbm.at[pl.ds(cbase0, NCHUNK)], dst_v)
        pltpu.sync_copy(ew_hbm.at[pl.ds(cbase0, NCHUNK)], ew_v)

        def fire_g(bufs, sem, cb):
            for j in range(K):
                pltpu.async_copy(h_hbm.at[src_v.at[cb + j]], bufs[j], sem)

        def drain_g(bufs, sem):
            for j in range(K):
                pltpu.make_async_copy(h_hbm.at[src_v.at[0]], bufs[j],
                                      sem).wait()

        def fire_s(bufs, sem, cb):
            for j in range(K):
                pltpu.async_copy(bufs[j], acc_s.at[dst_v.at[cb + j]], sem,
                                 add=True)

        def drain_s(bufs, sem):
            for j in range(K):
                pltpu.make_async_copy(bufs[j], acc_s.at[dst_v.at[0]],
                                      sem).wait()

        def compute(bufs, cb):
            for j in range(K):
                @plsc.parallel_loop(0, C, step=16, unroll=1)
                def group(gbase, j=j):
                    ew16 = ew_v[cb + j, pl.ds(gbase, 16)]
                    for e in range(16):
                        w = ew16.at[jnp.full((16,), e, jnp.int32)].get(
                            mode="promise_in_bounds")
                        for f in range(F // 16):
                            v = bufs[j][gbase + e, pl.ds(f * 16, 16)]
                            bufs[j][gbase + e, pl.ds(f * 16, 16)] = v * w

        fire_g(rows_a, gsem_a, 0)

        pltpu.sync_copy(zero_hbm.at[pl.ds(sid * RPT, RPT)],
                        acc_s.at[pl.ds(sid * RPT, RPT)])

        @pl.when(sid == NS - 1)
        def _():
            pltpu.sync_copy(zero_hbm.at[pl.ds(NS * RPT, RTAIL)],
                            acc_s.at[pl.ds(NS * RPT, RTAIL)])

        plsc.subcore_barrier()

        def pair(i, carry):
            a = 2 * i * K
            b = a + K

            @pl.when(i > 0)
            def _():
                drain_s(rows_b, ssem_b)

            fire_g(rows_b, gsem_b, b)
            drain_g(rows_a, gsem_a)
            compute(rows_a, a)
            fire_s(rows_a, ssem_a, a)
            drain_s(rows_a, ssem_a)
            fire_g(rows_a, gsem_a, a + 2 * K)
            drain_g(rows_b, gsem_b)
            compute(rows_b, b)
            fire_s(rows_b, ssem_b, b)
            return carry

        lax.fori_loop(0, NPAIR, pair, 0)
        eb = 2 * NPAIR * K
        drain_s(rows_b, ssem_b)
        drain_g(rows_a, gsem_a)
        compute(rows_a, eb)
        fire_s(rows_a, ssem_a, eb)
        drain_s(rows_a, ssem_a)
        plsc.subcore_barrier()
        pltpu.sync_copy(acc_s.at[pl.ds(sid * RPT, RPT)],
                        out_hbm.at[pl.ds(cid * N + sid * RPT, RPT)])

        @pl.when(sid == NS - 1)
        def _():
            pltpu.sync_copy(acc_s.at[pl.ds(NS * RPT, RTAIL)],
                            out_hbm.at[pl.ds(cid * N + NS * RPT, RTAIL)])

    return scat


def _mm_first(x, W):
    def body(x_ref, w_ref, o_ref):
        o_ref[...] = jnp.dot(x_ref[...], w_ref[...],
                             preferred_element_type=jnp.float32)

    Fi, Fo = W.shape
    return pl.pallas_call(
        body,
        grid=(N // BR,),
        in_specs=[pl.BlockSpec((BR, Fi), lambda i: (i, 0)),
                  pl.BlockSpec((Fi, Fo), lambda i: (0, 0))],
        out_specs=pl.BlockSpec((BR, Fo), lambda i: (i, 0)),
        out_shape=jax.ShapeDtypeStruct((N, Fo), jnp.float32),
    )(x, W)


def _fuse_layer(p, b, W):
    def body(p_ref, b_ref, w_ref, o_ref):
        s = p_ref[0] + p_ref[1] + b_ref[...]
        o_ref[...] = jnp.dot(_leaky(s), w_ref[...],
                             preferred_element_type=jnp.float32)

    F = p.shape[-1]
    Fo = W.shape[1]
    return pl.pallas_call(
        body,
        grid=(N // BR,),
        in_specs=[pl.BlockSpec((2, BR, F), lambda i: (0, i, 0)),
                  pl.BlockSpec((1, F), lambda i: (0, 0)),
                  pl.BlockSpec((F, Fo), lambda i: (0, 0))],
        out_specs=pl.BlockSpec((BR, Fo), lambda i: (i, 0)),
        out_shape=jax.ShapeDtypeStruct((N, Fo), jnp.float32),
    )(p, b, W)


def _tail(p, b, gamma, beta, batch3d, w1, b1, w2, b2, w3, b3):
    F = p.shape[-1]
    H = w1.shape[1]

    def body(p_ref, b_ref, g_ref, be_ref, bt_ref, w1r, b1r, w2r, b2r, w3r,
             b3r, o_ref, conv_s, st_s, pool_s):
        i = pl.program_id(0)

        @pl.when(i == 0)
        def _():
            st_s[...] = jnp.zeros_like(st_s)
            pool_s[...] = jnp.zeros_like(pool_s)

        @pl.when(i < NB)
        def _():
            s = p_ref[0] + p_ref[1] + b_ref[...]
            conv_s[pl.ds(i * BR, BR), :] = s
            st_s[0:1, :] = st_s[0:1, :] + jnp.sum(s, axis=0, keepdims=True)
            st_s[1:2, :] = st_s[1:2, :] + jnp.sum(s * s, axis=0,
                                                  keepdims=True)

        @pl.when((i >= NB) & (i < 2 * NB))
        def _():
            k = i - NB
            mean = st_s[0:1, :] * (1.0 / N)
            var = st_s[1:2, :] * (1.0 / N) - mean * mean
            inv = lax.rsqrt(var + 1e-5)
            s = conv_s[pl.ds(k * BR, BR), :]
            s = _leaky((s - mean) * inv * g_ref[...] + be_ref[...])
            sel = (bt_ref[0] ==
                   lax.broadcasted_iota(jnp.int32, (G, 1), 0)
                   ).astype(jnp.float32)
            pool_s[...] = pool_s[...] + jnp.dot(
                sel, s, preferred_element_type=jnp.float32)

        @pl.when(i == 2 * NB)
        def _():
            a = _leaky(jnp.dot(pool_s[...], w1r[...],
                               preferred_element_type=jnp.float32) + b1r[...])
            a = _leaky(jnp.dot(a, w2r[...],
                               preferred_element_type=jnp.float32) + b2r[...])
            a = _leaky(jnp.dot(a, w3r[...],
                               preferred_element_type=jnp.float32) + b3r[...])
            o_ref[...] = a

    clamp1 = lambda i: (0, jnp.minimum(i, NB - 1), 0)
    clamp2 = lambda i: (jnp.clip(i - NB, 0, NB - 1), 0, 0)
    const2 = lambda i: (0, 0)
    return pl.pallas_call(
        body,
        grid=(2 * NB + 1,),
        in_specs=[pl.BlockSpec((2, BR, F), clamp1),
                  pl.BlockSpec((1, F), const2),
                  pl.BlockSpec((1, F), const2),
                  pl.BlockSpec((1, F), const2),
                  pl.BlockSpec((1, 1, BR), clamp2),
                  pl.BlockSpec(w1.shape, const2),
                  pl.BlockSpec(b1.shape, const2),
                  pl.BlockSpec(w2.shape, const2),
                  pl.BlockSpec(b2.shape, const2),
                  pl.BlockSpec(w3.shape, const2),
                  pl.BlockSpec(b3.shape, const2)],
        out_specs=pl.BlockSpec((G, H), const2),
        out_shape=jax.ShapeDtypeStruct((G, H), jnp.float32),
        scratch_shapes=[pltpu.VMEM((N, F), jnp.float32),
                        pltpu.VMEM((8, F), jnp.float32),
                        pltpu.VMEM((G, F), jnp.float32)],
    )(p, b, gamma, beta, batch3d, w1, b1, w2, b2, w3, b3)


def _pad2(a, r, c):
    return jnp.pad(a, ((0, r - a.shape[0]), (0, c - a.shape[1])))


def kernel(x, edge_index, edge_weigth, batch, W1, b1, W2, b2, W3, b3, W4, b4,
           gamma, beta, fcw1, fcb1, fcw2, fcb2, fcw3, fcb3):
    src = edge_index[0]
    dst = edge_index[1]

    W4p = _pad2(W4, 64, 64)
    b4p = jnp.pad(b4, (0, 14)).reshape(1, 64)
    gammap = jnp.pad(gamma, (0, 14)).reshape(1, 64)
    betap = jnp.pad(beta, (0, 14)).reshape(1, 64)
    fw1 = _pad2(fcw1, 64, 128)
    fb1 = jnp.pad(fcb1, (0, 98)).reshape(1, 128)
    fw2 = _pad2(fcw2, 128, 128)
    fb2 = jnp.pad(fcb2, (0, 108)).reshape(1, 128)
    fw3 = _pad2(fcw3, 128, 128)
    fb3 = jnp.pad(fcb3, (0, 126)).reshape(1, 128)

    src2 = src.reshape(E // C, C)
    dst2 = dst.reshape(E // C, C)
    ew2 = edge_weigth.reshape(E // C, C)

    def scat(h, F):
        zeros = jnp.zeros((N, F), jnp.float32)
        p = _sc_scatter(F)(h, src2, dst2, ew2, zeros)
        return p.reshape(2, N, F)

    h1 = _mm_first(x, W1)
    p1 = scat(h1, 16)
    h2 = _fuse_layer(p1, b1.reshape(1, 16), W2)
    p2 = scat(h2, 32)
    h3 = _fuse_layer(p2, b2.reshape(1, 32), W3)
    p3 = scat(h3, 64)
    h4 = _fuse_layer(p3, b3.reshape(1, 64), W4p)
    p4 = scat(h4, 64)
    out = _tail(p4, b4p, gammap, betap, batch.reshape(NB, 1, BR),
                fw1, fb1, fw2, fb2, fw3, fb3)
    return out[:, :2]
